# Initial kernel scaffold; baseline (speedup 1.0000x reference)
import jax
import jax.numpy as jnp
from jax.experimental import pallas as pl


def kernel(x, hyperedge_index, hyperedge_attr, W_in, W_e, W_a, W_n, W_self, W_proj, W_dec):
    raise NotImplementedError("write your pallas kernel here")


# speedup vs baseline: 3.8380x; 3.8380x over previous
import functools

import jax
import jax.numpy as jnp
from jax import lax
from jax.experimental import pallas as pl
from jax.experimental.pallas import tpu as pltpu
from jax.experimental.pallas import tpu_sc as plsc

N_NODES = 10000
N_HEDGES = 10000
N_CONN = 320000
IN_DIM = 128
HIDDEN = 128
NUM_LAYERS = 3
N_MASK = 7000

NC = 2
NS = 16
NW = NC * NS
CHUNK = 80
CONN_PER_W = N_CONN // NW
N_CHUNKS = CONN_PER_W // CHUNK
NZ_TILES = 10
ROWS_PER_ZTILE = N_NODES // NZ_TILES

@functools.cache
def _mesh():
    return plsc.VectorSubcoreMesh(core_axis_name="c", subcore_axis_name="s")


def _seg_sum_sc(table, gidx, sidx, zeros128):
    @functools.partial(
        pl.kernel,
        out_type=jax.ShapeDtypeStruct((NC, N_NODES, HIDDEN), jnp.float32),
        mesh=_mesh(),
        scratch_types=[
            pltpu.VMEM((CHUNK,), jnp.int32),
            pltpu.VMEM((CHUNK,), jnp.int32),
            pltpu.VMEM((CHUNK, HIDDEN), jnp.float32),
            pltpu.VMEM_SHARED((N_NODES, HIDDEN), jnp.float32),
        ],
    )
    def k(table_hbm, gidx_hbm, sidx_hbm, zeros_hbm, out_hbm, gi_v, si_v, rows_v, accum):
        cid = lax.axis_index("c")
        sid = lax.axis_index("s")
        wid = cid * NS + sid
        rslice = pl.ds(sid * ROWS_PER_ZTILE, ROWS_PER_ZTILE)

        @pl.when(sid < NZ_TILES)
        def _():
            pltpu.sync_copy(zeros_hbm.at[rslice], accum.at[rslice])

        plsc.subcore_barrier()
        base = wid * CONN_PER_W

        @pl.loop(0, N_CHUNKS)
        def _(c):
            off = base + c * CHUNK
            pltpu.sync_copy(gidx_hbm.at[pl.ds(off, CHUNK)], gi_v)
            pltpu.sync_copy(sidx_hbm.at[pl.ds(off, CHUNK)], si_v)
            pltpu.sync_copy(table_hbm.at[gi_v], rows_v)
            pltpu.sync_copy(rows_v, accum.at[si_v], add=True)

        plsc.subcore_barrier()

        @pl.when(sid < NZ_TILES)
        def _():
            pltpu.sync_copy(accum.at[rslice], out_hbm.at[cid, rslice])

    return k(table, gidx, sidx, zeros128)


def _deg_sc(idx, zeros128):
    @functools.partial(
        pl.kernel,
        out_type=jax.ShapeDtypeStruct((NC, N_NODES, HIDDEN), jnp.float32),
        mesh=_mesh(),
        scratch_types=[
            pltpu.VMEM((CHUNK,), jnp.int32),
            pltpu.VMEM((CHUNK, HIDDEN), jnp.float32),
            pltpu.VMEM_SHARED((N_NODES, HIDDEN), jnp.float32),
        ],
    )
    def k(idx_hbm, zeros_hbm, out_hbm, i_v, ones_v, acc):
        cid = lax.axis_index("c")
        sid = lax.axis_index("s")
        wid = cid * NS + sid
        rslice = pl.ds(sid * ROWS_PER_ZTILE, ROWS_PER_ZTILE)

        @pl.when(sid < NZ_TILES)
        def _():
            pltpu.sync_copy(zeros_hbm.at[rslice], acc.at[rslice])

        @pl.loop(0, CHUNK)
        def _(i):
            @pl.loop(0, HIDDEN, step=16)
            def _(j):
                ones_v[i, pl.ds(j, 16)] = jnp.full((16,), 1.0, jnp.float32)

        plsc.subcore_barrier()
        base = wid * CONN_PER_W

        @pl.loop(0, N_CHUNKS)
        def _(c):
            off = base + c * CHUNK
            pltpu.sync_copy(idx_hbm.at[pl.ds(off, CHUNK)], i_v)
            pltpu.sync_copy(ones_v, acc.at[i_v], add=True)

        plsc.subcore_barrier()

        @pl.when(sid < NZ_TILES)
        def _():
            pltpu.sync_copy(acc.at[rslice], out_hbm.at[cid, rslice])

    return k(idx, zeros128)


def _elu(v):
    return jnp.where(v > 0.0, v, jnp.exp(jnp.minimum(v, 0.0)) - 1.0)


def _pre_tc(x, w0, wm, wu):
    def body(x_ref, w0_ref, wm_ref, wu_ref, o_ref):
        rows = lax.broadcasted_iota(jnp.int32, (N_NODES, 1), 0)
        m = (rows < N_MASK).astype(jnp.float32)
        xm = x_ref[...] * (1.0 - m)
        acc = jnp.dot(xm, w0_ref[...], preferred_element_type=jnp.float32)
        acc = acc + m * wm_ref[...] + (1.0 - m) * wu_ref[...]
        o_ref[...] = _elu(acc)

    return pl.pallas_call(
        body,
        out_shape=jax.ShapeDtypeStruct((N_NODES, HIDDEN), jnp.float32),
    )(x, w0, wm, wu)


def _dense_tc(partials, deg_partials, y, w_main, w_y):
    def body(p_ref, d_ref, y_ref, wm_ref, wy_ref, o_ref):
        p = p_ref[0] + p_ref[1]
        deg = jnp.maximum(d_ref[0, :, 0:1] + d_ref[1, :, 0:1], 1.0)
        acc = jnp.dot(p / deg, wm_ref[...], preferred_element_type=jnp.float32)
        acc = acc + jnp.dot(y_ref[...], wy_ref[...], preferred_element_type=jnp.float32)
        o_ref[...] = _elu(acc)

    return pl.pallas_call(
        body,
        out_shape=jax.ShapeDtypeStruct((N_NODES, HIDDEN), jnp.float32),
    )(partials, deg_partials, y, w_main, w_y)


def _out_tc(h, wp, wd):
    def body(h_ref, wp_ref, wd_ref, o_ref):
        z = jnp.dot(h_ref[...], wp_ref[...], preferred_element_type=jnp.float32)
        o_ref[...] = jnp.dot(z, wd_ref[...], preferred_element_type=jnp.float32)

    return pl.pallas_call(
        body,
        out_shape=jax.ShapeDtypeStruct((N_NODES, IN_DIM), jnp.float32),
    )(h, wp, wd)


def kernel(x, hyperedge_index, hyperedge_attr, W_in, W_e, W_a, W_n, W_self, W_proj, W_dec):
    node_idx = hyperedge_index[0]
    edge_idx = hyperedge_index[1]
    zeros128 = jnp.zeros((N_NODES, HIDDEN), jnp.float32)

    degn_p = _deg_sc(node_idx, zeros128)
    dege_p = _deg_sc(edge_idx, zeros128)
    h = _pre_tc(x, W_in[:IN_DIM], W_in[IN_DIM:IN_DIM + 1], W_in[IN_DIM + 1:IN_DIM + 2])

    for l in range(NUM_LAYERS):
        e_p = _seg_sum_sc(h, node_idx, edge_idx, zeros128)
        e = _dense_tc(e_p, dege_p, hyperedge_attr, W_e[l], W_a[l])
        n_p = _seg_sum_sc(e, edge_idx, node_idx, zeros128)
        h = _dense_tc(n_p, degn_p, h, W_n[l], W_self[l])

    return _out_tc(h, W_proj, W_dec)


# speedup vs baseline: 7.1319x; 1.8582x over previous
import functools

import jax
import jax.numpy as jnp
from jax import lax
from jax.experimental import pallas as pl
from jax.experimental.pallas import tpu as pltpu
from jax.experimental.pallas import tpu_sc as plsc

N_NODES = 10000
N_HEDGES = 10000
N_CONN = 320000
IN_DIM = 128
HIDDEN = 128
NUM_LAYERS = 3
N_MASK = 7000

NC = 2
NS = 16
NW = NC * NS
CHUNK = 80
CONN_PER_W = N_CONN // NW
N_CHUNKS = CONN_PER_W // CHUNK
NZ_TILES = 10
ROWS_PER_ZTILE = N_NODES // NZ_TILES

@functools.cache
def _mesh():
    return plsc.VectorSubcoreMesh(core_axis_name="c", subcore_axis_name="s")


def _seg_sum_sc(table, gidx, sidx, zeros128):
    @functools.partial(
        pl.kernel,
        out_type=jax.ShapeDtypeStruct((NC, N_NODES, HIDDEN), jnp.float32),
        mesh=_mesh(),
        scratch_types=[
            pltpu.VMEM((4, CHUNK), jnp.int32),
            pltpu.VMEM((4, CHUNK), jnp.int32),
            pltpu.VMEM((2, CHUNK, HIDDEN), jnp.float32),
            pltpu.VMEM_SHARED((N_NODES, HIDDEN), jnp.float32),
            pltpu.SemaphoreType.DMA((4,)),
            pltpu.SemaphoreType.DMA((2,)),
            pltpu.SemaphoreType.DMA((2,)),
        ],
    )
    def k(table_hbm, gidx_hbm, sidx_hbm, zeros_hbm, out_hbm,
          gi_v, si_v, rows_v, accum, isem, gsem, ssem):
        cid = lax.axis_index("c")
        sid = lax.axis_index("s")
        wid = cid * NS + sid
        rslice = pl.ds(sid * ROWS_PER_ZTILE, ROWS_PER_ZTILE)

        @pl.when(sid < NZ_TILES)
        def _():
            pltpu.sync_copy(zeros_hbm.at[rslice], accum.at[rslice])

        plsc.subcore_barrier()
        base = wid * CONN_PER_W

        def idx_start(c, s):
            off = base + c * CHUNK
            pltpu.async_copy(gidx_hbm.at[pl.ds(off, CHUNK)], gi_v.at[s], isem.at[s])
            pltpu.async_copy(sidx_hbm.at[pl.ds(off, CHUNK)], si_v.at[s], isem.at[s])

        def idx_wait(s):
            pltpu.make_async_copy(gidx_hbm.at[pl.ds(base, CHUNK)], gi_v.at[s], isem.at[s]).wait()
            pltpu.make_async_copy(sidx_hbm.at[pl.ds(base, CHUNK)], si_v.at[s], isem.at[s]).wait()

        def gather_start(s4, s2):
            pltpu.async_copy(table_hbm.at[gi_v.at[s4]], rows_v.at[s2], gsem.at[s2])

        def gather_wait(s4, s2):
            pltpu.make_async_copy(table_hbm.at[gi_v.at[s4]], rows_v.at[s2], gsem.at[s2]).wait()

        def scatter_start(s2, s4):
            pltpu.async_copy(rows_v.at[s2], accum.at[si_v.at[s4]], ssem.at[s2], add=True)

        def scatter_wait(s2, s4):
            pltpu.make_async_copy(rows_v.at[s2], accum.at[si_v.at[s4]], ssem.at[s2]).wait()

        idx_start(0, 0)
        idx_start(1, 1)
        idx_wait(0)
        gather_start(0, 0)

        @pl.loop(0, N_CHUNKS)
        def _(c):
            b2 = lax.rem(c, 2)
            b2n = lax.rem(c + 1, 2)
            b4 = lax.rem(c, 4)
            gather_wait(b4, b2)
            scatter_start(b2, b4)

            @pl.when(c + 1 < N_CHUNKS)
            def _():
                s4 = lax.rem(c + 1, 4)
                idx_wait(s4)

                @pl.when(c >= 1)
                def _():
                    scatter_wait(b2n, lax.rem(c + 3, 4))

                gather_start(s4, b2n)

            @pl.when(c + 2 < N_CHUNKS)
            def _():
                idx_start(c + 2, lax.rem(c + 2, 4))

        scatter_wait(lax.rem(N_CHUNKS - 1, 2), lax.rem(N_CHUNKS - 1, 4))
        scatter_wait(lax.rem(N_CHUNKS - 2, 2), lax.rem(N_CHUNKS - 2, 4))
        plsc.subcore_barrier()

        @pl.when(sid < NZ_TILES)
        def _():
            pltpu.sync_copy(accum.at[rslice], out_hbm.at[cid, rslice])

    return k(table, gidx, sidx, zeros128)


def _deg_sc(idx, zeros128):
    @functools.partial(
        pl.kernel,
        out_type=jax.ShapeDtypeStruct((NC, N_NODES, HIDDEN), jnp.float32),
        mesh=_mesh(),
        scratch_types=[
            pltpu.VMEM((4, CHUNK), jnp.int32),
            pltpu.VMEM((CHUNK, HIDDEN), jnp.float32),
            pltpu.VMEM_SHARED((N_NODES, HIDDEN), jnp.float32),
            pltpu.SemaphoreType.DMA((4,)),
            pltpu.SemaphoreType.DMA((2,)),
        ],
    )
    def k(idx_hbm, zeros_hbm, out_hbm, i_v, ones_v, acc, isem, ssem):
        cid = lax.axis_index("c")
        sid = lax.axis_index("s")
        wid = cid * NS + sid
        rslice = pl.ds(sid * ROWS_PER_ZTILE, ROWS_PER_ZTILE)

        @pl.when(sid < NZ_TILES)
        def _():
            pltpu.sync_copy(zeros_hbm.at[rslice], acc.at[rslice])

        @pl.loop(0, CHUNK)
        def _(i):
            @pl.loop(0, HIDDEN, step=16)
            def _(j):
                ones_v[i, pl.ds(j, 16)] = jnp.full((16,), 1.0, jnp.float32)

        plsc.subcore_barrier()
        base = wid * CONN_PER_W

        def idx_start(c, s):
            pltpu.async_copy(idx_hbm.at[pl.ds(base + c * CHUNK, CHUNK)], i_v.at[s], isem.at[s])

        def idx_wait(s):
            pltpu.make_async_copy(idx_hbm.at[pl.ds(base, CHUNK)], i_v.at[s], isem.at[s]).wait()

        def scatter_start(s4, s2):
            pltpu.async_copy(ones_v, acc.at[i_v.at[s4]], ssem.at[s2], add=True)

        def scatter_wait(s4, s2):
            pltpu.make_async_copy(ones_v, acc.at[i_v.at[s4]], ssem.at[s2]).wait()

        idx_start(0, 0)
        idx_start(1, 1)

        @pl.loop(0, N_CHUNKS)
        def _(c):
            b2 = lax.rem(c, 2)
            b4 = lax.rem(c, 4)

            @pl.when(c >= 2)
            def _():
                scatter_wait(lax.rem(c + 2, 4), b2)

            idx_wait(b4)
            scatter_start(b4, b2)

            @pl.when(c + 2 < N_CHUNKS)
            def _():
                idx_start(c + 2, lax.rem(c + 2, 4))

        scatter_wait(lax.rem(N_CHUNKS - 1, 4), lax.rem(N_CHUNKS - 1, 2))
        scatter_wait(lax.rem(N_CHUNKS - 2, 4), lax.rem(N_CHUNKS - 2, 2))
        plsc.subcore_barrier()

        @pl.when(sid < NZ_TILES)
        def _():
            pltpu.sync_copy(acc.at[rslice], out_hbm.at[cid, rslice])

    return k(idx, zeros128)


def _elu(v):
    return jnp.where(v > 0.0, v, jnp.exp(jnp.minimum(v, 0.0)) - 1.0)


def _pre_tc(x, w0, wm, wu):
    def body(x_ref, w0_ref, wm_ref, wu_ref, o_ref):
        rows = lax.broadcasted_iota(jnp.int32, (N_NODES, 1), 0)
        m = (rows < N_MASK).astype(jnp.float32)
        xm = x_ref[...] * (1.0 - m)
        acc = jnp.dot(xm, w0_ref[...], preferred_element_type=jnp.float32)
        acc = acc + m * wm_ref[...] + (1.0 - m) * wu_ref[...]
        o_ref[...] = _elu(acc)

    return pl.pallas_call(
        body,
        out_shape=jax.ShapeDtypeStruct((N_NODES, HIDDEN), jnp.float32),
    )(x, w0, wm, wu)


def _dense_tc(partials, deg_partials, y, w_main, w_y):
    def body(p_ref, d_ref, y_ref, wm_ref, wy_ref, o_ref):
        p = p_ref[0] + p_ref[1]
        deg = jnp.maximum(d_ref[0, :, 0:1] + d_ref[1, :, 0:1], 1.0)
        acc = jnp.dot(p / deg, wm_ref[...], preferred_element_type=jnp.float32)
        acc = acc + jnp.dot(y_ref[...], wy_ref[...], preferred_element_type=jnp.float32)
        o_ref[...] = _elu(acc)

    return pl.pallas_call(
        body,
        out_shape=jax.ShapeDtypeStruct((N_NODES, HIDDEN), jnp.float32),
    )(partials, deg_partials, y, w_main, w_y)


def _out_tc(h, wp, wd):
    def body(h_ref, wp_ref, wd_ref, o_ref):
        z = jnp.dot(h_ref[...], wp_ref[...], preferred_element_type=jnp.float32)
        o_ref[...] = jnp.dot(z, wd_ref[...], preferred_element_type=jnp.float32)

    return pl.pallas_call(
        body,
        out_shape=jax.ShapeDtypeStruct((N_NODES, IN_DIM), jnp.float32),
    )(h, wp, wd)


def kernel(x, hyperedge_index, hyperedge_attr, W_in, W_e, W_a, W_n, W_self, W_proj, W_dec):
    node_idx = hyperedge_index[0]
    edge_idx = hyperedge_index[1]
    zeros128 = jnp.zeros((N_NODES, HIDDEN), jnp.float32)

    degn_p = _deg_sc(node_idx, zeros128)
    dege_p = _deg_sc(edge_idx, zeros128)
    h = _pre_tc(x, W_in[:IN_DIM], W_in[IN_DIM:IN_DIM + 1], W_in[IN_DIM + 1:IN_DIM + 2])

    for l in range(NUM_LAYERS):
        e_p = _seg_sum_sc(h, node_idx, edge_idx, zeros128)
        e = _dense_tc(e_p, dege_p, hyperedge_attr, W_e[l], W_a[l])
        n_p = _seg_sum_sc(e, edge_idx, node_idx, zeros128)
        h = _dense_tc(n_p, degn_p, h, W_n[l], W_self[l])

    return _out_tc(h, W_proj, W_dec)
